# baseline (speedup 1.0000x reference)
import functools

import jax
import jax.numpy as jnp
from jax import lax
from jax.experimental import pallas as pl
from jax.experimental.pallas import tpu as pltpu
from jax.experimental.pallas import tpu_sc as plsc

L = 16
CHUNK = 128


def _sc_complex_score(ent_real, ent_img, rel_cat, h, r, t):
    B = h.shape[0]
    D = ent_real.shape[1]
    n_rel = rel_cat.shape[0]
    info = plsc.get_sparse_core_info()
    nc, ns = info.num_cores, info.num_subcores
    nw = nc * ns
    b_per_w = B // nw
    n_chunks = b_per_w // CHUNK
    n_dvec = D // L
    groups = CHUNK // L
    mesh = plsc.VectorSubcoreMesh(core_axis_name="c", subcore_axis_name="s")

    @functools.partial(
        pl.kernel,
        mesh=mesh,
        compiler_params=pltpu.CompilerParams(needs_layout_passes=False),
        out_type=jax.ShapeDtypeStruct((B,), jnp.float32),
        scratch_types=[
            pltpu.VMEM((n_chunks, CHUNK), jnp.int32),
            pltpu.VMEM((n_chunks, CHUNK), jnp.int32),
            pltpu.VMEM((n_chunks, CHUNK), jnp.int32),
            pltpu.VMEM((n_chunks, CHUNK), jnp.float32),
            pltpu.VMEM((CHUNK, D), jnp.float32),
            pltpu.VMEM((CHUNK, D), jnp.float32),
            pltpu.VMEM((CHUNK, D), jnp.float32),
            pltpu.VMEM((CHUNK, D), jnp.float32),
            pltpu.VMEM((CHUNK, 2 * D), jnp.float32),
            pltpu.VMEM((b_per_w,), jnp.float32),
            pltpu.SemaphoreType.DMA,
        ],
    )
    def k(ent_real_h, ent_img_h, rel_cat_h, h_h, r_h, t_h, out_h,
          h_v, t_v, rm_v, sign, hr, hi, tr, ti, rc, out_v, sem):
        wid = lax.axis_index("s") * nc + lax.axis_index("c")
        base = wid * b_per_w

        icps = []
        for j in range(n_chunks):
            off = base + j * CHUNK
            icps.append(pltpu.async_copy(h_h.at[pl.ds(off, CHUNK)], h_v.at[j], sem))
            icps.append(pltpu.async_copy(t_h.at[pl.ds(off, CHUNK)], t_v.at[j], sem))
            icps.append(pltpu.async_copy(r_h.at[pl.ds(off, CHUNK)], rm_v.at[j], sem))
        for cp in icps:
            cp.wait()

        for j in range(n_chunks):
            for g in range(groups):
                sl = pl.ds(g * L, L)
                rv = rm_v[j, sl]
                ge = rv >= n_rel
                rm_v[j, sl] = rv - jnp.where(ge, n_rel, 0)
                sign[j, sl] = jnp.where(ge, -1.0, 1.0).astype(jnp.float32)

        iota = lax.iota(jnp.int32, L)
        for j in range(n_chunks):
            rel_cp = pltpu.async_copy(rel_cat_h.at[rm_v.at[j]], rc, sem)

            def issue_body(g, carry, j=j):
                sl = pl.ds(g * L, L)
                hv = h_v[j, sl]
                tv = t_v[j, sl]
                for kk in range(L):
                    ih = hv[kk]
                    it = tv[kk]
                    e = g * L + kk
                    pltpu.async_copy(ent_real_h.at[ih], hr.at[e], sem)
                    pltpu.async_copy(ent_img_h.at[ih], hi.at[e], sem)
                    pltpu.async_copy(ent_real_h.at[it], tr.at[e], sem)
                    pltpu.async_copy(ent_img_h.at[it], ti.at[e], sem)
                return carry

            lax.fori_loop(0, groups, issue_body, 0)

            rel_cp.wait()
            dummy = ent_real_h.at[pl.ds(0, CHUNK)]
            for buf in (hr, hi, tr, ti):
                pltpu.make_async_copy(dummy, buf, sem).wait()

            def group_body(g, carry, j=j):
                score_a = jnp.zeros((L,), jnp.float32)
                score_b = jnp.zeros((L,), jnp.float32)
                for kk in range(L):
                    e = g * L + kk
                    acc_a = jnp.zeros((L,), jnp.float32)
                    acc_b = jnp.zeros((L,), jnp.float32)
                    for c in range(n_dvec):
                        sl = pl.ds(c * L, L)
                        r1 = hr[e, sl]
                        i1 = hi[e, sl]
                        r3 = tr[e, sl]
                        i3 = ti[e, sl]
                        r2 = rc[e, sl]
                        i2 = rc[e, pl.ds(D + c * L, L)]
                        acc_a = acc_a + r2 * (r1 * r3 + i1 * i3)
                        acc_b = acc_b + i2 * (r1 * i3 - i1 * r3)
                    m = iota == kk
                    score_a = jnp.where(m, jnp.broadcast_to(jnp.sum(acc_a), (L,)), score_a)
                    score_b = jnp.where(m, jnp.broadcast_to(jnp.sum(acc_b), (L,)), score_b)
                sv = sign[j, pl.ds(g * L, L)]
                out_v[pl.ds(j * CHUNK + g * L, L)] = score_a + sv * score_b
                return carry

            lax.fori_loop(0, groups, group_body, 0)

        pltpu.sync_copy(out_v, out_h.at[pl.ds(base, b_per_w)])

    return k(ent_real, ent_img, rel_cat, h, r, t)


def kernel(ent_real, ent_img, rel_real, rel_img, h, r, t):
    rel_cat = jnp.concatenate([rel_real, rel_img], axis=1)
    return _sc_complex_score(
        ent_real, ent_img, rel_cat,
        h.astype(jnp.int32), r.astype(jnp.int32), t.astype(jnp.int32),
    )
